# Initial kernel scaffold; baseline (speedup 1.0000x reference)
import jax
import jax.numpy as jnp
from jax.experimental import pallas as pl


def kernel(x, edge_index, W1, b1, W2):
    raise NotImplementedError("write your pallas kernel here")


# speedup vs baseline: 8.8685x; 8.8685x over previous
import functools

import jax
import jax.numpy as jnp
from jax import lax
from jax.experimental import pallas as pl
from jax.experimental.pallas import tpu as pltpu
from jax.experimental.pallas import tpu_sc as plsc

_NC = 2
_NS = 16
_NW = _NC * _NS
_CHUNK = 128


def _mm1(x, w1):
    n, d_in = x.shape
    d_hid = w1.shape[1]
    blk = 1000
    grid = n // blk

    def body(x_ref, w_ref, o_ref):
        o_ref[...] = jnp.dot(x_ref[...], w_ref[...],
                             preferred_element_type=jnp.float32)

    return pl.pallas_call(
        body,
        grid=(grid,),
        in_specs=[
            pl.BlockSpec((blk, d_in), lambda i: (i, 0)),
            pl.BlockSpec((d_in, d_hid), lambda i: (0, 0)),
        ],
        out_specs=pl.BlockSpec((blk, d_hid), lambda i: (i, 0)),
        out_shape=jax.ShapeDtypeStruct((n, d_hid), jnp.float32),
    )(x, w1)


def _sc_gather_scatter_add(srcr, dstr, h, zrows, acc_rows, stripe, k_chunks):
    d_hid = h.shape[1]
    mesh = plsc.VectorSubcoreMesh(core_axis_name="c", subcore_axis_name="s")

    @functools.partial(
        pl.kernel,
        mesh=mesh,
        compiler_params=pltpu.CompilerParams(use_tc_tiling_on_sc=False),
        out_type=jax.ShapeDtypeStruct((_NC, acc_rows, d_hid), jnp.float32),
        scratch_types=[
            pltpu.VMEM((k_chunks, _CHUNK), jnp.int32),
            pltpu.VMEM((k_chunks, _CHUNK), jnp.int32),
            pltpu.VMEM((_CHUNK, d_hid), jnp.float32),
            pltpu.VMEM_SHARED((acc_rows, d_hid), jnp.float32),
            pltpu.SemaphoreType.DMA,
        ],
    )
    def k(srcr_hbm, dstr_hbm, h_hbm, z_hbm, out_hbm, sidx, didx, rows, acc, sem):
        c = lax.axis_index("c")
        s = lax.axis_index("s")
        g = c * _NS + s
        pltpu.sync_copy(z_hbm, acc.at[pl.ds(s * stripe, stripe)])
        pltpu.sync_copy(srcr_hbm.at[g], sidx)
        pltpu.sync_copy(dstr_hbm.at[g], didx)
        plsc.subcore_barrier()

        @pl.loop(0, k_chunks)
        def _(j):
            pltpu.async_copy(h_hbm.at[sidx.at[j]], rows, sem).wait()
            pltpu.sync_copy(rows, acc.at[didx.at[j]], add=True)

        plsc.subcore_barrier()
        pltpu.sync_copy(acc.at[pl.ds(s * stripe, stripe)],
                        out_hbm.at[c, pl.ds(s * stripe, stripe)])

    return k(srcr, dstr, h, zrows)


def _tc2(p0, p1, b1r, w2, n):
    d_hid = p0.shape[1]
    d_out = w2.shape[1]
    blk = 400
    grid = n // blk

    def body(p0_ref, p1_ref, b_ref, w_ref, h1_ref, h2_ref):
        acc = p0_ref[...] + p1_ref[...] + b_ref[...]
        h1_ref[...] = acc
        h2_ref[...] = jnp.dot(acc, w_ref[...],
                              preferred_element_type=jnp.float32)

    return pl.pallas_call(
        body,
        grid=(grid,),
        in_specs=[
            pl.BlockSpec((blk, d_hid), lambda i: (i, 0)),
            pl.BlockSpec((blk, d_hid), lambda i: (i, 0)),
            pl.BlockSpec((1, d_hid), lambda i: (0, 0)),
            pl.BlockSpec((d_hid, d_out), lambda i: (0, 0)),
        ],
        out_specs=[
            pl.BlockSpec((blk, d_hid), lambda i: (i, 0)),
            pl.BlockSpec((blk, d_out), lambda i: (i, 0)),
        ],
        out_shape=[
            jax.ShapeDtypeStruct((n, d_hid), jnp.float32),
            jax.ShapeDtypeStruct((n, d_out), jnp.float32),
        ],
    )(p0, p1, b1r, w2)


def kernel(x, edge_index, W1, b1, W2):
    n, d_hid = x.shape[0], W1.shape[1]
    e = edge_index.shape[1]

    per_op = _NW * _CHUNK
    k_chunks = -(-e // per_op)
    ep = k_chunks * per_op
    stripe = -(-(n + 1) // (8 * _NS)) * 8
    acc_rows = stripe * _NS

    src = edge_index[0]
    dst = edge_index[1]
    pad = ep - e
    src_p = jnp.concatenate([src, jnp.zeros((pad,), jnp.int32)])
    dst_p = jnp.concatenate([dst, jnp.full((pad,), n, jnp.int32)])
    srcr = src_p.reshape(_NW, k_chunks, _CHUNK)
    dstr = dst_p.reshape(_NW, k_chunks, _CHUNK)
    zrows = jnp.zeros((stripe, d_hid), jnp.float32)

    h = _mm1(x, W1)
    part = _sc_gather_scatter_add(srcr, dstr, h, zrows,
                                  acc_rows, stripe, k_chunks)
    h1, h2 = _tc2(part[0], part[1], b1.reshape(1, d_hid), W2, n)
    return (h1, h2)


# speedup vs baseline: 14.1184x; 1.5920x over previous
import functools

import jax
import jax.numpy as jnp
from jax import lax
from jax.experimental import pallas as pl
from jax.experimental.pallas import tpu as pltpu
from jax.experimental.pallas import tpu_sc as plsc

_NC = 2
_NS = 16
_NW = _NC * _NS
_CHUNK = 128


def _mm1(x, w1):
    n, d_in = x.shape
    d_hid = w1.shape[1]
    blk = 1000
    grid = n // blk

    def body(x_ref, w_ref, o_ref):
        o_ref[...] = jnp.dot(x_ref[...], w_ref[...],
                             preferred_element_type=jnp.float32)

    return pl.pallas_call(
        body,
        grid=(grid,),
        in_specs=[
            pl.BlockSpec((blk, d_in), lambda i: (i, 0)),
            pl.BlockSpec((d_in, d_hid), lambda i: (0, 0)),
        ],
        out_specs=pl.BlockSpec((blk, d_hid), lambda i: (i, 0)),
        out_shape=jax.ShapeDtypeStruct((n, d_hid), jnp.float32),
    )(x, w1)


def _sc_gather_scatter_add(srcr, dstr, h, zrows, acc_rows, stripe, k_chunks):
    n, d_hid = h.shape
    h_stripe = n // _NS
    mesh = plsc.VectorSubcoreMesh(core_axis_name="c", subcore_axis_name="s")

    @functools.partial(
        pl.kernel,
        mesh=mesh,
        compiler_params=pltpu.CompilerParams(use_tc_tiling_on_sc=False),
        out_type=jax.ShapeDtypeStruct((_NC, acc_rows, d_hid), jnp.float32),
        scratch_types=[
            pltpu.VMEM((k_chunks, _CHUNK), jnp.int32),
            pltpu.VMEM((k_chunks, _CHUNK), jnp.int32),
            pltpu.VMEM((_CHUNK, d_hid), jnp.float32),
            pltpu.VMEM((_CHUNK, d_hid), jnp.float32),
            pltpu.VMEM_SHARED((n, d_hid), jnp.float32),
            pltpu.VMEM_SHARED((acc_rows, d_hid), jnp.float32),
            pltpu.SemaphoreType.DMA,
            pltpu.SemaphoreType.DMA,
        ],
    )
    def k(srcr_hbm, dstr_hbm, h_hbm, z_hbm, out_hbm,
          sidx, didx, rows0, rows1, hsh, acc, sem0, sem1):
        c = lax.axis_index("c")
        s = lax.axis_index("s")
        g = c * _NS + s
        pltpu.sync_copy(z_hbm, acc.at[pl.ds(s * stripe, stripe)])
        pltpu.sync_copy(h_hbm.at[pl.ds(s * h_stripe, h_stripe)],
                        hsh.at[pl.ds(s * h_stripe, h_stripe)])
        pltpu.sync_copy(srcr_hbm.at[g], sidx)
        pltpu.sync_copy(dstr_hbm.at[g], didx)
        plsc.subcore_barrier()

        pltpu.async_copy(hsh.at[sidx.at[0]], rows0, sem0)
        pltpu.async_copy(hsh.at[sidx.at[1]], rows1, sem1)

        @pl.loop(0, k_chunks, step=2)
        def _(j):
            pltpu.make_async_copy(hsh.at[sidx.at[j]], rows0, sem0).wait()
            pltpu.sync_copy(rows0, acc.at[didx.at[j]], add=True)

            @pl.when(j + 2 < k_chunks)
            def _():
                pltpu.async_copy(hsh.at[sidx.at[j + 2]], rows0, sem0)

            pltpu.make_async_copy(hsh.at[sidx.at[j + 1]], rows1, sem1).wait()
            pltpu.sync_copy(rows1, acc.at[didx.at[j + 1]], add=True)

            @pl.when(j + 3 < k_chunks)
            def _():
                pltpu.async_copy(hsh.at[sidx.at[j + 3]], rows1, sem1)

        plsc.subcore_barrier()
        pltpu.sync_copy(acc.at[pl.ds(s * stripe, stripe)],
                        out_hbm.at[c, pl.ds(s * stripe, stripe)])

    return k(srcr, dstr, h, zrows)


def _tc2(p0, p1, b1r, w2, n):
    d_hid = p0.shape[1]
    d_out = w2.shape[1]
    blk = 400
    grid = n // blk

    def body(p0_ref, p1_ref, b_ref, w_ref, h1_ref, h2_ref):
        acc = p0_ref[...] + p1_ref[...] + b_ref[...]
        h1_ref[...] = acc
        h2_ref[...] = jnp.dot(acc, w_ref[...],
                              preferred_element_type=jnp.float32)

    return pl.pallas_call(
        body,
        grid=(grid,),
        in_specs=[
            pl.BlockSpec((blk, d_hid), lambda i: (i, 0)),
            pl.BlockSpec((blk, d_hid), lambda i: (i, 0)),
            pl.BlockSpec((1, d_hid), lambda i: (0, 0)),
            pl.BlockSpec((d_hid, d_out), lambda i: (0, 0)),
        ],
        out_specs=[
            pl.BlockSpec((blk, d_hid), lambda i: (i, 0)),
            pl.BlockSpec((blk, d_out), lambda i: (i, 0)),
        ],
        out_shape=[
            jax.ShapeDtypeStruct((n, d_hid), jnp.float32),
            jax.ShapeDtypeStruct((n, d_out), jnp.float32),
        ],
    )(p0, p1, b1r, w2)


def kernel(x, edge_index, W1, b1, W2):
    n, d_hid = x.shape[0], W1.shape[1]
    e = edge_index.shape[1]

    per_op = _NW * _CHUNK
    k_chunks = -(-e // per_op)
    k_chunks += k_chunks % 2
    ep = k_chunks * per_op
    stripe = -(-(n + 1) // (8 * _NS)) * 8
    acc_rows = stripe * _NS

    src = edge_index[0]
    dst = edge_index[1]
    pad = ep - e
    src_p = jnp.concatenate([src, jnp.zeros((pad,), jnp.int32)])
    dst_p = jnp.concatenate([dst, jnp.full((pad,), n, jnp.int32)])
    srcr = src_p.reshape(_NW, k_chunks, _CHUNK)
    dstr = dst_p.reshape(_NW, k_chunks, _CHUNK)
    zrows = jnp.zeros((stripe, d_hid), jnp.float32)

    h = _mm1(x, W1)
    part = _sc_gather_scatter_add(srcr, dstr, h, zrows,
                                  acc_rows, stripe, k_chunks)
    h1, h2 = _tc2(part[0], part[1], b1.reshape(1, d_hid), W2, n)
    return (h1, h2)


# speedup vs baseline: 16.3979x; 1.1615x over previous
import functools

import jax
import jax.numpy as jnp
from jax import lax
from jax.experimental import pallas as pl
from jax.experimental.pallas import tpu as pltpu
from jax.experimental.pallas import tpu_sc as plsc

_NC = 2
_NS = 16
_NW = _NC * _NS
_CHUNK = 128


def _mm1(x, w1):
    n, d_in = x.shape
    d_hid = w1.shape[1]
    blk = 2000
    grid = n // blk

    def body(x_ref, w_ref, o_ref):
        o_ref[...] = jnp.dot(x_ref[...], w_ref[...],
                             preferred_element_type=jnp.float32)

    return pl.pallas_call(
        body,
        grid=(grid,),
        in_specs=[
            pl.BlockSpec((blk, d_in), lambda i: (i, 0)),
            pl.BlockSpec((d_in, d_hid), lambda i: (0, 0)),
        ],
        out_specs=pl.BlockSpec((blk, d_hid), lambda i: (i, 0)),
        out_shape=jax.ShapeDtypeStruct((n, d_hid), jnp.float32),
    )(x, w1)


def _sc_gather_scatter_add(edges, h, zrows, stripe, kbase, kextra, kmax):
    n, d_hid = h.shape
    h_stripe = n // _NS
    mesh = plsc.VectorSubcoreMesh(core_axis_name="c", subcore_axis_name="s")

    @functools.partial(
        pl.kernel,
        mesh=mesh,
        compiler_params=pltpu.CompilerParams(use_tc_tiling_on_sc=False),
        out_type=jax.ShapeDtypeStruct((_NC, _NS * stripe, d_hid), jnp.float32),
        scratch_types=[
            pltpu.VMEM((kmax, _CHUNK), jnp.int32),
            pltpu.VMEM((kmax, _CHUNK), jnp.int32),
            pltpu.VMEM((_CHUNK, d_hid), jnp.float32),
            pltpu.VMEM((_CHUNK, d_hid), jnp.float32),
            pltpu.VMEM_SHARED((n, d_hid), jnp.float32),
            pltpu.VMEM_SHARED((_NS * stripe, d_hid), jnp.float32),
            pltpu.SemaphoreType.DMA,
            pltpu.SemaphoreType.DMA,
            pltpu.SemaphoreType.DMA,
            pltpu.SemaphoreType.DMA,
        ],
    )
    def k(e_hbm, h_hbm, z_hbm, out_hbm,
          sidx, didx, rows0, rows1, hsh, acc, gs0, gs1, ss0, ss1):
        c = lax.axis_index("c")
        s = lax.axis_index("s")
        g = c * _NS + s
        kw = kbase + jnp.where(g < kextra, 1, 0)
        cb = g * kbase + jnp.minimum(g, kextra)
        pltpu.sync_copy(z_hbm, acc.at[pl.ds(s * stripe, stripe)])
        pltpu.sync_copy(h_hbm.at[pl.ds(s * h_stripe, h_stripe)],
                        hsh.at[pl.ds(s * h_stripe, h_stripe)])
        pltpu.sync_copy(e_hbm.at[0, pl.ds(cb, kbase)], sidx.at[pl.ds(0, kbase)])
        pltpu.sync_copy(e_hbm.at[1, pl.ds(cb, kbase)], didx.at[pl.ds(0, kbase)])

        @pl.when(g < kextra)
        def _():
            pltpu.sync_copy(e_hbm.at[0, pl.ds(cb + kbase, 1)],
                            sidx.at[pl.ds(kbase, 1)])
            pltpu.sync_copy(e_hbm.at[1, pl.ds(cb + kbase, 1)],
                            didx.at[pl.ds(kbase, 1)])

        plsc.subcore_barrier()

        pltpu.async_copy(hsh.at[sidx.at[0]], rows0, gs0)

        @pl.when(1 < kw)
        def _():
            pltpu.async_copy(hsh.at[sidx.at[1]], rows1, gs1)

        @pl.loop(0, kw, step=2)
        def _(j):
            pltpu.make_async_copy(hsh.at[sidx.at[j]], rows0, gs0).wait()
            pltpu.async_copy(rows0, acc.at[didx.at[j]], ss0, add=True)

            @pl.when(j + 1 < kw)
            def _():
                pltpu.make_async_copy(hsh.at[sidx.at[j + 1]], rows1, gs1).wait()
                pltpu.async_copy(rows1, acc.at[didx.at[j + 1]], ss1, add=True)

            @pl.when(j + 2 < kw)
            def _():
                pltpu.make_async_copy(rows0, acc.at[didx.at[j]], ss0).wait()
                pltpu.async_copy(hsh.at[sidx.at[j + 2]], rows0, gs0)

            @pl.when(j + 3 < kw)
            def _():
                pltpu.make_async_copy(rows1, acc.at[didx.at[j + 1]], ss1).wait()
                pltpu.async_copy(hsh.at[sidx.at[j + 3]], rows1, gs1)

        pltpu.make_async_copy(rows0, acc.at[didx.at[0]], ss0).wait()
        pltpu.make_async_copy(rows1, acc.at[didx.at[0]], ss1).wait()

        plsc.subcore_barrier()
        pltpu.sync_copy(acc.at[pl.ds(s * stripe, stripe)],
                        out_hbm.at[c, pl.ds(s * stripe, stripe)])

    return k(edges, h, zrows)


def _tc2(part, b1r, w2, n):
    d_hid = part.shape[2]
    d_out = w2.shape[1]
    blk = 400
    grid = n // blk

    def body(p0_ref, p1_ref, b_ref, w_ref, h1_ref, h2_ref):
        acc = p0_ref[0] + p1_ref[0] + b_ref[...]
        h1_ref[...] = acc
        h2_ref[...] = jnp.dot(acc, w_ref[...],
                              preferred_element_type=jnp.float32)

    return pl.pallas_call(
        body,
        grid=(grid,),
        in_specs=[
            pl.BlockSpec((1, blk, d_hid), lambda i: (0, i, 0)),
            pl.BlockSpec((1, blk, d_hid), lambda i: (1, i, 0)),
            pl.BlockSpec((1, d_hid), lambda i: (0, 0)),
            pl.BlockSpec((d_hid, d_out), lambda i: (0, 0)),
        ],
        out_specs=[
            pl.BlockSpec((blk, d_hid), lambda i: (i, 0)),
            pl.BlockSpec((blk, d_out), lambda i: (i, 0)),
        ],
        out_shape=[
            jax.ShapeDtypeStruct((n, d_hid), jnp.float32),
            jax.ShapeDtypeStruct((n, d_out), jnp.float32),
        ],
    )(part, part, b1r, w2)


def kernel(x, edge_index, W1, b1, W2):
    n, d_hid = x.shape[0], W1.shape[1]
    e = edge_index.shape[1]

    total_chunks = e // _CHUNK
    kbase = total_chunks // _NW
    kextra = total_chunks % _NW
    kmax = kbase + (1 if kextra else 0)
    stripe = n // _NS

    edges = edge_index.reshape(2, total_chunks, _CHUNK)
    zrows = jnp.zeros((stripe, d_hid), jnp.float32)

    h = _mm1(x, W1)
    part = _sc_gather_scatter_add(edges, h, zrows, stripe, kbase, kextra, kmax)
    h1, h2 = _tc2(part, b1.reshape(1, d_hid), W2, n)
    return (h1, h2)


# speedup vs baseline: 20.2575x; 1.2354x over previous
import functools

import jax
import jax.numpy as jnp
from jax import lax
from jax.experimental import pallas as pl
from jax.experimental.pallas import tpu as pltpu
from jax.experimental.pallas import tpu_sc as plsc

_NC = 2
_NS = 16
_NW = _NC * _NS
_CHUNK = 128
_LANES = 128


def _mm1(x, w1):
    n, d_in = x.shape
    d_hid = w1.shape[1]
    blk = 2000
    grid = n // blk

    def body(x_ref, w_ref, o_ref):
        res = jnp.dot(x_ref[...], w_ref[...],
                      preferred_element_type=jnp.float32)
        o_ref[...] = jnp.pad(res, ((0, 0), (0, _LANES - d_hid)))

    return pl.pallas_call(
        body,
        grid=(grid,),
        in_specs=[
            pl.BlockSpec((blk, d_in), lambda i: (i, 0)),
            pl.BlockSpec((d_in, d_hid), lambda i: (0, 0)),
        ],
        out_specs=pl.BlockSpec((blk, _LANES), lambda i: (i, 0)),
        out_shape=jax.ShapeDtypeStruct((n, _LANES), jnp.float32),
    )(x, w1)


def _sc_gather_scatter_add(edges, h, zrows, d_hid, stripe, kbase, kextra, kmax):
    n = h.shape[0]
    h_stripe = n // _NS
    mesh = plsc.VectorSubcoreMesh(core_axis_name="c", subcore_axis_name="s")

    @functools.partial(
        pl.kernel,
        mesh=mesh,
        compiler_params=pltpu.CompilerParams(use_tc_tiling_on_sc=False),
        out_type=jax.ShapeDtypeStruct((_NC, n, _LANES), jnp.float32),
        scratch_types=[
            pltpu.VMEM((kmax, _CHUNK), jnp.int32),
            pltpu.VMEM((kmax, _CHUNK), jnp.int32),
            pltpu.VMEM((_CHUNK, d_hid), jnp.float32),
            pltpu.VMEM((_CHUNK, d_hid), jnp.float32),
            pltpu.VMEM_SHARED((n, d_hid), jnp.float32),
            pltpu.VMEM_SHARED((_NS * (n // _NS), d_hid), jnp.float32),
            pltpu.SemaphoreType.DMA,
            pltpu.SemaphoreType.DMA,
            pltpu.SemaphoreType.DMA,
            pltpu.SemaphoreType.DMA,
        ],
    )
    def k(e_hbm, h_hbm, z_hbm, out_hbm,
          sidx, didx, rows0, rows1, hsh, acc, gs0, gs1, ss0, ss1):
        c = lax.axis_index("c")
        s = lax.axis_index("s")
        g = c * _NS + s
        kw = kbase + jnp.where(g < kextra, 1, 0)
        cb = g * kbase + jnp.minimum(g, kextra)
        pltpu.sync_copy(z_hbm, acc.at[pl.ds(s * stripe, stripe)])
        pltpu.sync_copy(h_hbm.at[pl.ds(s * h_stripe, h_stripe), pl.ds(0, d_hid)],
                        hsh.at[pl.ds(s * h_stripe, h_stripe)])
        pltpu.sync_copy(e_hbm.at[0, pl.ds(cb, kbase)], sidx.at[pl.ds(0, kbase)])
        pltpu.sync_copy(e_hbm.at[1, pl.ds(cb, kbase)], didx.at[pl.ds(0, kbase)])

        @pl.when(g < kextra)
        def _():
            pltpu.sync_copy(e_hbm.at[0, pl.ds(cb + kbase, 1)],
                            sidx.at[pl.ds(kbase, 1)])
            pltpu.sync_copy(e_hbm.at[1, pl.ds(cb + kbase, 1)],
                            didx.at[pl.ds(kbase, 1)])

        plsc.subcore_barrier()

        pltpu.async_copy(hsh.at[sidx.at[0]], rows0, gs0)

        @pl.when(1 < kw)
        def _():
            pltpu.async_copy(hsh.at[sidx.at[1]], rows1, gs1)

        @pl.loop(0, kw, step=2)
        def _(j):
            pltpu.make_async_copy(hsh.at[sidx.at[j]], rows0, gs0).wait()
            pltpu.async_copy(rows0, acc.at[didx.at[j]], ss0, add=True)

            @pl.when(j + 1 < kw)
            def _():
                pltpu.make_async_copy(hsh.at[sidx.at[j + 1]], rows1, gs1).wait()
                pltpu.async_copy(rows1, acc.at[didx.at[j + 1]], ss1, add=True)

            @pl.when(j + 2 < kw)
            def _():
                pltpu.make_async_copy(rows0, acc.at[didx.at[j]], ss0).wait()
                pltpu.async_copy(hsh.at[sidx.at[j + 2]], rows0, gs0)

            @pl.when(j + 3 < kw)
            def _():
                pltpu.make_async_copy(rows1, acc.at[didx.at[j + 1]], ss1).wait()
                pltpu.async_copy(hsh.at[sidx.at[j + 3]], rows1, gs1)

        pltpu.make_async_copy(rows0, acc.at[didx.at[0]], ss0).wait()
        pltpu.make_async_copy(rows1, acc.at[didx.at[0]], ss1).wait()

        plsc.subcore_barrier()
        pltpu.sync_copy(acc.at[pl.ds(s * stripe, stripe)],
                        out_hbm.at[c, pl.ds(s * stripe, stripe), pl.ds(0, d_hid)])

    return k(edges, h, zrows)


def _tc2(part, b1r, w2, n, d_hid):
    d_out = w2.shape[1]
    blk = 2000
    grid = n // blk

    def body(p0_ref, p1_ref, b_ref, w_ref, h1_ref, h2_ref):
        acc = (p0_ref[0, :, :d_hid] + p1_ref[0, :, :d_hid]) + b_ref[...]
        h1_ref[...] = acc
        h2_ref[...] = jnp.dot(acc, w_ref[...],
                              preferred_element_type=jnp.float32)

    return pl.pallas_call(
        body,
        grid=(grid,),
        in_specs=[
            pl.BlockSpec((1, blk, _LANES), lambda i: (0, i, 0)),
            pl.BlockSpec((1, blk, _LANES), lambda i: (1, i, 0)),
            pl.BlockSpec((1, d_hid), lambda i: (0, 0)),
            pl.BlockSpec((d_hid, d_out), lambda i: (0, 0)),
        ],
        out_specs=[
            pl.BlockSpec((blk, d_hid), lambda i: (i, 0)),
            pl.BlockSpec((blk, d_out), lambda i: (i, 0)),
        ],
        out_shape=[
            jax.ShapeDtypeStruct((n, d_hid), jnp.float32),
            jax.ShapeDtypeStruct((n, d_out), jnp.float32),
        ],
    )(part, part, b1r, w2)


def kernel(x, edge_index, W1, b1, W2):
    n, d_hid = x.shape[0], W1.shape[1]
    e = edge_index.shape[1]

    total_chunks = e // _CHUNK
    kbase = total_chunks // _NW
    kextra = total_chunks % _NW
    kmax = kbase + (1 if kextra else 0)
    stripe = n // _NS

    edges = edge_index.reshape(2, total_chunks, _CHUNK)
    zrows = jnp.zeros((stripe, d_hid), jnp.float32)

    h = _mm1(x, W1)
    part = _sc_gather_scatter_add(edges, h, zrows, d_hid,
                                  stripe, kbase, kextra, kmax)
    h1, h2 = _tc2(part, b1.reshape(1, d_hid), W2, n, d_hid)
    return (h1, h2)
